# baseline (speedup 1.0000x reference)
import jax
import jax.numpy as jnp
from jax import lax
from jax.experimental import pallas as pl
from jax.experimental.pallas import tpu as pltpu
from jax.experimental.pallas import tpu_sc as plsc

N = 10000
E = 160000
F_IN = 256
FH = 128
H = 512

NSC = 2
NT = 16
CKA = 256
NCH = 40
EP = NT * NCH * CKA
EPT = EP // NT
CKD = 1024
SL = 640


def _sc_body(xv_hbm, src_hbm, dst_hbm, w_hbm, s2_hbm, dinv_hbm,
             S_sp, deg_sp, dinv_sp, dinv_v, degsl, dstd, wd,
             srcc0, dstc0, wc0, srcc1, dstc1, wc1, rows, gsem, ssem):
    c = lax.axis_index("c")
    s = lax.axis_index("s")
    dbase = s * EPT

    def _zdeg(i, _):
        degsl[pl.ds(i * 16, 16)] = jnp.zeros((16,), jnp.float32)
        return ()
    lax.fori_loop(0, SL // 16, _zdeg, ())
    soff = jnp.minimum(s * SL, N - SL)
    pltpu.sync_copy(degsl, deg_sp.at[pl.ds(soff, SL)])

    def _zrow(r, _):
        for f in range(8):
            rows[r, pl.ds(f * 16, 16)] = jnp.zeros((16,), jnp.float32)
        return ()
    lax.fori_loop(0, CKA, _zrow, ())
    for o in (0, 128, 256, 384, 512):
        pltpu.sync_copy(rows.at[pl.ds(0, 128)],
                        S_sp.at[pl.ds(soff + o, 128)])
    plsc.subcore_barrier()

    def _dchunk(g, _):
        off = dbase + g * CKD
        pltpu.sync_copy(dst_hbm.at[pl.ds(off, CKD)], dstd)
        pltpu.sync_copy(w_hbm.at[pl.ds(off, CKD)], wd)
        pltpu.sync_copy(wd, deg_sp.at[dstd], add=True)
        return ()
    lax.fori_loop(0, EPT // CKD, _dchunk, ())
    plsc.subcore_barrier()

    pltpu.sync_copy(deg_sp.at[pl.ds(soff, SL)], degsl)
    def _newton(i, _):
        d = degsl[pl.ds(i * 16, 16)] + 1.0
        half = 0.5 * d
        ib = lax.bitcast_convert_type(d, jnp.int32)
        ib = jnp.int32(0x5F3759DF) - lax.shift_right_logical(ib, 1)
        r = lax.bitcast_convert_type(ib, jnp.float32)
        for _ in range(4):
            r = r * (1.5 - half * r * r)
        degsl[pl.ds(i * 16, 16)] = r
        return ()
    lax.fori_loop(0, SL // 16, _newton, ())
    pltpu.sync_copy(degsl, dinv_sp.at[pl.ds(soff, SL)])
    @pl.when(c == 0)
    def _():
        pltpu.sync_copy(degsl, dinv_hbm.at[pl.ds(soff, SL)])
    plsc.subcore_barrier()
    pltpu.sync_copy(dinv_sp, dinv_v)

    sets = ((srcc0, dstc0, wc0), (srcc1, dstc1, wc1))

    def _load(g, st):
        srcc, dstc, wc = st
        off = dbase + g * CKA
        pltpu.sync_copy(src_hbm.at[pl.ds(off, CKA)], srcc)
        pltpu.sync_copy(dst_hbm.at[pl.ds(off, CKA)], dstc)
        pltpu.sync_copy(w_hbm.at[pl.ds(off, CKA)], wc)
        @plsc.parallel_loop(0, CKA // 16, unroll=2)
        def _idx(i):
            sl = pl.ds(i * 16, 16)
            sv = srcc[sl]
            wc[sl] = wc[sl] * plsc.load_gather(dinv_v, [sv])
            srcc[sl] = sv * 2 + c
    def _wait_scatter(st):
        pltpu.make_async_copy(rows, S_sp.at[st[1]], ssem).wait()

    def _half(g, stA, stB):
        @pl.when(g > 0)
        def _():
            _wait_scatter(stB)
        gat = pltpu.async_copy(xv_hbm.at[stA[0]], rows, gsem)
        @pl.when(g + 1 < NCH)
        def _():
            _load(g + 1, stB)
        gat.wait()
        cv = stA[2]
        @plsc.parallel_loop(0, CKA, unroll=4)
        def _scale(e):
            cs = plsc.load_gather(cv, [jnp.full((16,), e, jnp.int32)])
            for f in range(8):
                sl = (e, pl.ds(f * 16, 16))
                rows[sl] = rows[sl] * cs
        pltpu.async_copy(rows, S_sp.at[stA[1]], ssem, add=True)

    _load(0, sets[0])
    def _pair(g2, _):
        g = g2 * 2
        _half(g, sets[0], sets[1])
        _half(g + 1, sets[1], sets[0])
        return ()
    lax.fori_loop(0, NCH // 2, _pair, ())
    _wait_scatter(sets[1])
    plsc.subcore_barrier()

    pltpu.sync_copy(S_sp.at[pl.ds(soff, SL)],
                    s2_hbm.at[c].at[pl.ds(soff, SL)])


def _sc_aggregate(xv, srcf, dstf, wf):
    mesh = plsc.VectorSubcoreMesh(core_axis_name="c", subcore_axis_name="s")
    return pl.kernel(
        _sc_body,
        out_type=[
            jax.ShapeDtypeStruct((NSC, N, FH), jnp.float32),
            jax.ShapeDtypeStruct((N,), jnp.float32),
        ],
        mesh=mesh,
        compiler_params=pltpu.CompilerParams(needs_layout_passes=False),
        scratch_types=[
            pltpu.VMEM_SHARED((N, FH), jnp.float32),
            pltpu.VMEM_SHARED((N,), jnp.float32),
            pltpu.VMEM_SHARED((N,), jnp.float32),
            pltpu.VMEM((N,), jnp.float32),
            pltpu.VMEM((SL,), jnp.float32),
            pltpu.VMEM((CKD,), jnp.int32),
            pltpu.VMEM((CKD,), jnp.float32),
            pltpu.VMEM((CKA,), jnp.int32),
            pltpu.VMEM((CKA,), jnp.int32),
            pltpu.VMEM((CKA,), jnp.float32),
            pltpu.VMEM((CKA,), jnp.int32),
            pltpu.VMEM((CKA,), jnp.int32),
            pltpu.VMEM((CKA,), jnp.float32),
            pltpu.VMEM((CKA, FH), jnp.float32),
            pltpu.SemaphoreType.DMA,
            pltpu.SemaphoreType.DMA,
        ],
    )(xv, srcf, dstf, wf)


BND = 2000


def _tc_body(s2, x, dinv, W1r, b1r, gar, Wgr, bgr, Wl1r, bl1r, Wl2r, bl2r,
             out, acc):
    i = pl.program_id(0)

    @pl.when(i == 0)
    def _():
        acc[...] = jnp.zeros_like(acc)

    dv = dinv[...]
    t = jnp.concatenate([s2[0], s2[1]], axis=1)
    agg = dv * (t + dv * x[...])
    h = jnp.dot(agg, W1r[...], preferred_element_type=jnp.float32) + b1r[...]
    h = jnp.maximum(h, 0.0)
    acc[...] += jnp.sum(h, axis=0, keepdims=True)

    @pl.when(i == pl.num_programs(0) - 1)
    def _():
        hm = acc[...] / N
        g = jnp.dot(gar[...], Wgr[...], preferred_element_type=jnp.float32)
        g = jnp.maximum(g + bgr[...], 0.0)
        z = jnp.concatenate([hm, g], axis=1)
        z1 = jnp.dot(z, Wl1r[...], preferred_element_type=jnp.float32)
        z1 = jnp.maximum(z1 + bl1r[...], 0.0)
        z2 = jnp.dot(z1, Wl2r[...], preferred_element_type=jnp.float32)
        z2 = z2 + bl2r[...]
        m = jnp.max(z2, axis=1, keepdims=True)
        lse = m + jnp.log(jnp.sum(jnp.exp(z2 - m), axis=1, keepdims=True))
        out[...] = z2 - lse


def _tc_head(s2, x, dinv2, W1, b1, ga, Wg, bg, Wl1, bl1, Wl2, bl2):
    nsteps = N // BND
    full = lambda shape: pl.BlockSpec(shape, lambda i: tuple(0 for _ in shape))
    return pl.pallas_call(
        _tc_body,
        grid=(nsteps,),
        in_specs=[
            pl.BlockSpec((NSC, BND, FH), lambda i: (0, i, 0)),
            pl.BlockSpec((BND, F_IN), lambda i: (i, 0)),
            pl.BlockSpec((BND, 1), lambda i: (i, 0)),
            full((F_IN, H)),
            full((1, H)),
            full((1, 64)),
            full((64, H)),
            full((1, H)),
            full((2 * H, H)),
            full((1, H)),
            full((H, 2)),
            full((1, 2)),
        ],
        out_specs=pl.BlockSpec((1, 2), lambda i: (0, 0)),
        out_shape=jax.ShapeDtypeStruct((1, 2), jnp.float32),
        scratch_shapes=[pltpu.VMEM((1, H), jnp.float32)],
    )(s2, x, dinv2, W1, b1, ga, Wg, bg, Wl1, bl1, Wl2, bl2)


def kernel(x, edge_index, edge_attr, graph_attr, W1, b1, Wg, bg, Wl1, bl1,
           Wl2, bl2):
    if graph_attr.ndim == 1:
        graph_attr = graph_attr[None, :]
    xv = x.reshape(NSC * N, FH)
    pad = EP - E
    srcf = jnp.pad(edge_index[0], (0, pad))
    dstf = jnp.pad(edge_index[1], (0, pad))
    wf = jnp.pad(edge_attr, (0, pad))
    s2, dinv = _sc_aggregate(xv, srcf, dstf, wf)
    return _tc_head(s2, x, dinv.reshape(N, 1), W1, b1.reshape(1, H),
                    graph_attr, Wg, bg.reshape(1, H), Wl1, bl1.reshape(1, H),
                    Wl2, bl2.reshape(1, 2))


# speedup vs baseline: 1.2746x; 1.2746x over previous
import jax
import jax.numpy as jnp
from jax import lax
from jax.experimental import pallas as pl
from jax.experimental.pallas import tpu as pltpu
from jax.experimental.pallas import tpu_sc as plsc

N = 10000
E = 160000
F_IN = 256
FH = 128
H = 512

NSC = 2
NT = 16
CKD = 1000
CKA = 200
EPT = E // NT
SL = 640


def _sc_body(x2_hbm, src_hbm, dst_hbm, w_hbm, s2_hbm, dinv_hbm,
             S_sp, deg_sp, dinv_sp, dinv_v, degsl, dstd, wd,
             srca, dsta, wa, cv, rows, sem):
    c = lax.axis_index("c")
    s = lax.axis_index("s")

    def _zdeg(i, _):
        degsl[pl.ds(i * 16, 16)] = jnp.zeros((16,), jnp.float32)
        return ()
    lax.fori_loop(0, SL // 16, _zdeg, ())
    soff = jnp.minimum(s * SL, N - SL)
    pltpu.sync_copy(degsl, deg_sp.at[pl.ds(soff, SL)])

    def _zrow(r, _):
        for f in range(8):
            rows[r, pl.ds(f * 16, 16)] = jnp.zeros((16,), jnp.float32)
        return ()
    lax.fori_loop(0, CKA, _zrow, ())
    for o in (0, 160, 320, 480):
        pltpu.sync_copy(rows.at[pl.ds(0, 160)],
                        S_sp.at[pl.ds(soff + o, 160)])
    plsc.subcore_barrier()

    dbase = s * EPT
    def _dchunk(g, _):
        off = dbase + g * CKD
        pltpu.sync_copy(dst_hbm.at[pl.ds(off, CKD)], dstd)
        pltpu.sync_copy(w_hbm.at[pl.ds(off, CKD)], wd)
        pltpu.sync_copy(wd, deg_sp.at[dstd], add=True)
        return ()
    lax.fori_loop(0, EPT // CKD, _dchunk, ())
    plsc.subcore_barrier()

    pltpu.sync_copy(deg_sp.at[pl.ds(soff, SL)], degsl)
    def _newton(i, _):
        d = degsl[pl.ds(i * 16, 16)] + 1.0
        half = 0.5 * d
        ib = lax.bitcast_convert_type(d, jnp.int32)
        ib = jnp.int32(0x5F3759DF) - lax.shift_right_logical(ib, 1)
        r = lax.bitcast_convert_type(ib, jnp.float32)
        for _ in range(4):
            r = r * (1.5 - half * r * r)
        degsl[pl.ds(i * 16, 16)] = r
        return ()
    lax.fori_loop(0, SL // 16, _newton, ())
    pltpu.sync_copy(degsl, dinv_sp.at[pl.ds(soff, SL)])
    @pl.when(c == 0)
    def _():
        pltpu.sync_copy(degsl, dinv_hbm.at[pl.ds(soff, SL)])
    plsc.subcore_barrier()
    pltpu.sync_copy(dinv_sp, dinv_v)

    def _chunk(g, _):
        off = dbase + g * CKA
        pltpu.sync_copy(src_hbm.at[pl.ds(off, CKA)], srca)
        pltpu.sync_copy(dst_hbm.at[pl.ds(off, CKA)], dsta)
        pltpu.sync_copy(w_hbm.at[pl.ds(off, CKA)], wa)
        gat = pltpu.async_copy(x2_hbm.at[c].at[srca], rows, sem)
        def _cb(i, _):
            sv = srca[pl.ds(i * 16, 16)]
            dvec = plsc.load_gather(dinv_v, [sv])
            cv[pl.ds(i * 16, 16)] = wa[pl.ds(i * 16, 16)] * dvec
            return ()
        lax.fori_loop(0, CKA // 16, _cb, ())
        o = CKA - 16
        sv = srca[pl.ds(o, 16)]
        dvec = plsc.load_gather(dinv_v, [sv])
        cv[pl.ds(o, 16)] = wa[pl.ds(o, 16)] * dvec
        gat.wait()
        @plsc.parallel_loop(0, CKA, unroll=4)
        def _scale(e):
            cs = plsc.load_gather(cv, [jnp.full((16,), e, jnp.int32)])
            for f in range(8):
                sl = (e, pl.ds(f * 16, 16))
                rows[sl] = rows[sl] * cs
        pltpu.sync_copy(rows, S_sp.at[dsta], add=True)
        return ()
    lax.fori_loop(0, EPT // CKA, _chunk, ())
    plsc.subcore_barrier()

    pltpu.sync_copy(S_sp.at[pl.ds(soff, SL)],
                    s2_hbm.at[c].at[pl.ds(soff, SL)])


def _sc_aggregate(x2, src, dst, w):
    mesh = plsc.VectorSubcoreMesh(core_axis_name="c", subcore_axis_name="s")
    return pl.kernel(
        _sc_body,
        out_type=[
            jax.ShapeDtypeStruct((NSC, N, FH), jnp.float32),
            jax.ShapeDtypeStruct((N,), jnp.float32),
        ],
        mesh=mesh,
        compiler_params=pltpu.CompilerParams(needs_layout_passes=False),
        scratch_types=[
            pltpu.VMEM_SHARED((N, FH), jnp.float32),
            pltpu.VMEM_SHARED((N,), jnp.float32),
            pltpu.VMEM_SHARED((N,), jnp.float32),
            pltpu.VMEM((N,), jnp.float32),
            pltpu.VMEM((SL,), jnp.float32),
            pltpu.VMEM((CKD,), jnp.int32),
            pltpu.VMEM((CKD,), jnp.float32),
            pltpu.VMEM((CKA,), jnp.int32),
            pltpu.VMEM((CKA,), jnp.int32),
            pltpu.VMEM((CKA,), jnp.float32),
            pltpu.VMEM((CKA,), jnp.float32),
            pltpu.VMEM((CKA, FH), jnp.float32),
            pltpu.SemaphoreType.DMA,
        ],
    )(x2, src, dst, w)


BND = 2000


def _tc_body(s2, x, dinv, W1r, b1r, gar, Wgr, bgr, Wl1r, bl1r, Wl2r, bl2r,
             out, acc):
    i = pl.program_id(0)

    @pl.when(i == 0)
    def _():
        acc[...] = jnp.zeros_like(acc)

    dv = dinv[...]
    t = jnp.concatenate([s2[0], s2[1]], axis=1)
    agg = dv * (t + dv * x[...])
    h = jnp.dot(agg, W1r[...], preferred_element_type=jnp.float32) + b1r[...]
    h = jnp.maximum(h, 0.0)
    acc[...] += jnp.sum(h, axis=0, keepdims=True)

    @pl.when(i == pl.num_programs(0) - 1)
    def _():
        hm = acc[...] / N
        g = jnp.dot(gar[...], Wgr[...], preferred_element_type=jnp.float32)
        g = jnp.maximum(g + bgr[...], 0.0)
        z = jnp.concatenate([hm, g], axis=1)
        z1 = jnp.dot(z, Wl1r[...], preferred_element_type=jnp.float32)
        z1 = jnp.maximum(z1 + bl1r[...], 0.0)
        z2 = jnp.dot(z1, Wl2r[...], preferred_element_type=jnp.float32)
        z2 = z2 + bl2r[...]
        m = jnp.max(z2, axis=1, keepdims=True)
        lse = m + jnp.log(jnp.sum(jnp.exp(z2 - m), axis=1, keepdims=True))
        out[...] = z2 - lse


def _tc_head(s2, x, dinv2, W1, b1, ga, Wg, bg, Wl1, bl1, Wl2, bl2):
    nsteps = N // BND
    full = lambda shape: pl.BlockSpec(shape, lambda i: tuple(0 for _ in shape))
    return pl.pallas_call(
        _tc_body,
        grid=(nsteps,),
        in_specs=[
            pl.BlockSpec((NSC, BND, FH), lambda i: (0, i, 0)),
            pl.BlockSpec((BND, F_IN), lambda i: (i, 0)),
            pl.BlockSpec((BND, 1), lambda i: (i, 0)),
            full((F_IN, H)),
            full((1, H)),
            full((1, 64)),
            full((64, H)),
            full((1, H)),
            full((2 * H, H)),
            full((1, H)),
            full((H, 2)),
            full((1, 2)),
        ],
        out_specs=pl.BlockSpec((1, 2), lambda i: (0, 0)),
        out_shape=jax.ShapeDtypeStruct((1, 2), jnp.float32),
        scratch_shapes=[pltpu.VMEM((1, H), jnp.float32)],
    )(s2, x, dinv2, W1, b1, ga, Wg, bg, Wl1, bl1, Wl2, bl2)


def kernel(x, edge_index, edge_attr, graph_attr, W1, b1, Wg, bg, Wl1, bl1,
           Wl2, bl2):
    if graph_attr.ndim == 1:
        graph_attr = graph_attr[None, :]
    src = edge_index[0]
    dst = edge_index[1]
    x2 = jnp.swapaxes(x.reshape(N, NSC, FH), 0, 1)
    s2, dinv = _sc_aggregate(x2, src, dst, edge_attr)
    return _tc_head(s2, x, dinv.reshape(N, 1), W1, b1.reshape(1, H),
                    graph_attr, Wg, bg.reshape(1, H), Wl1, bl1.reshape(1, H),
                    Wl2, bl2.reshape(1, 2))


# speedup vs baseline: 1.2898x; 1.0119x over previous
import jax
import jax.numpy as jnp
from jax import lax
from jax.experimental import pallas as pl
from jax.experimental.pallas import tpu as pltpu
from jax.experimental.pallas import tpu_sc as plsc

N = 10000
E = 160000
F_IN = 256
FH = 128
H = 512

NSC = 2
NT = 16
CKD = 1000
EPT = E // NT
CKA = 160
NCH = 63
EPT2 = NCH * CKA
E2 = NT * EPT2
SL = 640


def _deg_body(dst_hbm, w_hbm, deg_hbm, deg_sp, degsl, dstd, wd):
    c = lax.axis_index("c")
    s = lax.axis_index("s")
    soff = jnp.minimum(s * SL, N - SL)

    @pl.when(c == 0)
    def _():
        def _zdeg(i, _):
            degsl[pl.ds(i * 16, 16)] = jnp.zeros((16,), jnp.float32)
            return ()
        lax.fori_loop(0, SL // 16, _zdeg, ())
        pltpu.sync_copy(degsl, deg_sp.at[pl.ds(soff, SL)])
    plsc.subcore_barrier()

    @pl.when(c == 0)
    def _():
        dbase = s * EPT
        def _dchunk(g, _):
            off = dbase + g * CKD
            pltpu.sync_copy(dst_hbm.at[pl.ds(off, CKD)], dstd)
            pltpu.sync_copy(w_hbm.at[pl.ds(off, CKD)], wd)
            pltpu.sync_copy(wd, deg_sp.at[dstd], add=True)
            return ()
        lax.fori_loop(0, EPT // CKD, _dchunk, ())
    plsc.subcore_barrier()

    @pl.when(c == 0)
    def _():
        pltpu.sync_copy(deg_sp.at[pl.ds(soff, SL)], degsl)
        pltpu.sync_copy(degsl, deg_hbm.at[pl.ds(soff, SL)])


def _sc_deg(dst, w):
    mesh = plsc.VectorSubcoreMesh(core_axis_name="c", subcore_axis_name="s")
    return pl.kernel(
        _deg_body,
        out_type=[jax.ShapeDtypeStruct((N,), jnp.float32)],
        mesh=mesh,
        compiler_params=pltpu.CompilerParams(needs_layout_passes=False),
        scratch_types=[
            pltpu.VMEM_SHARED((N,), jnp.float32),
            pltpu.VMEM((SL,), jnp.float32),
            pltpu.VMEM((CKD,), jnp.int32),
            pltpu.VMEM((CKD,), jnp.float32),
        ],
    )(dst, w)[0]


def _agg_body(y2_hbm, src_hbm, dst_hbm, w_hbm, s2_hbm, S_sp,
              srca0, dsta0, wa0, rows0, gsem0, ssem0,
              srca1, dsta1, wa1, rows1, gsem1, ssem1):
    c = lax.axis_index("c")
    s = lax.axis_index("s")
    dbase = s * EPT2
    soff = jnp.minimum(s * SL, N - SL)

    def _zrow(r, _):
        for f in range(8):
            rows0[r, pl.ds(f * 16, 16)] = jnp.zeros((16,), jnp.float32)
        return ()
    lax.fori_loop(0, CKA, _zrow, ())
    for o in (0, 160, 320, 480):
        pltpu.sync_copy(rows0, S_sp.at[pl.ds(soff + o, CKA)])
    plsc.subcore_barrier()

    sets = ((srca0, dsta0, wa0, rows0, gsem0, ssem0),
            (srca1, dsta1, wa1, rows1, gsem1, ssem1))

    def _load(g, st):
        srca, dsta, wa = st[0], st[1], st[2]
        off = dbase + g * CKA
        pltpu.sync_copy(src_hbm.at[pl.ds(off, CKA)], srca)
        pltpu.sync_copy(dst_hbm.at[pl.ds(off, CKA)], dsta)
        pltpu.sync_copy(w_hbm.at[pl.ds(off, CKA)], wa)

    def _fire_gather(st):
        pltpu.async_copy(y2_hbm.at[c].at[st[0]], st[3], st[4])

    def _wait_gather(st):
        pltpu.make_async_copy(y2_hbm.at[c].at[st[0]], st[3], st[4]).wait()

    def _wait_scatter(st):
        pltpu.make_async_copy(st[3], S_sp.at[st[1]], st[5]).wait()

    def _half(g, A, B):
        _wait_gather(A)
        @pl.when(g > 0)
        def _():
            _wait_scatter(B)
        @pl.when(g + 1 < NCH)
        def _():
            _load(g + 1, B)
            _fire_gather(B)
        wa, rows = A[2], A[3]
        @plsc.parallel_loop(0, CKA, unroll=4)
        def _scale(e):
            cs = plsc.load_gather(wa, [jnp.full((16,), e, jnp.int32)])
            for f in range(8):
                sl = (e, pl.ds(f * 16, 16))
                rows[sl] = rows[sl] * cs
        pltpu.async_copy(rows, S_sp.at[A[1]], A[5], add=True)

    _load(0, sets[0])
    _fire_gather(sets[0])
    def _pair(g2, _):
        g = g2 * 2
        _half(g, sets[0], sets[1])
        _half(g + 1, sets[1], sets[0])
        return ()
    lax.fori_loop(0, (NCH - 1) // 2, _pair, ())
    _half(NCH - 1, sets[0], sets[1])
    _wait_scatter(sets[0])
    plsc.subcore_barrier()

    pltpu.sync_copy(S_sp.at[pl.ds(soff, SL)],
                    s2_hbm.at[c].at[pl.ds(soff, SL)])


def _sc_agg(y2, srcp, dstp, wp):
    mesh = plsc.VectorSubcoreMesh(core_axis_name="c", subcore_axis_name="s")
    buf = lambda: [
        pltpu.VMEM((CKA,), jnp.int32),
        pltpu.VMEM((CKA,), jnp.int32),
        pltpu.VMEM((CKA,), jnp.float32),
        pltpu.VMEM((CKA, FH), jnp.float32),
        pltpu.SemaphoreType.DMA,
        pltpu.SemaphoreType.DMA,
    ]
    return pl.kernel(
        _agg_body,
        out_type=[jax.ShapeDtypeStruct((NSC, N, FH), jnp.float32)],
        mesh=mesh,
        compiler_params=pltpu.CompilerParams(needs_layout_passes=False),
        scratch_types=[pltpu.VMEM_SHARED((N, FH), jnp.float32)]
        + buf() + buf(),
    )(y2, srcp, dstp, wp)[0]


BN = 2000


def _pre_body(deg, x, y2, dinv2):
    dv = lax.rsqrt(1.0 + deg[...])
    y = x[...] * dv
    y2[0] = y[:, :FH]
    y2[1] = y[:, FH:]
    dinv2[...] = dv


def _tc_pre(deg2, x):
    return pl.pallas_call(
        _pre_body,
        grid=(N // BN,),
        in_specs=[
            pl.BlockSpec((BN, 1), lambda i: (i, 0)),
            pl.BlockSpec((BN, F_IN), lambda i: (i, 0)),
        ],
        out_specs=[
            pl.BlockSpec((NSC, BN, FH), lambda i: (0, i, 0)),
            pl.BlockSpec((BN, 1), lambda i: (i, 0)),
        ],
        out_shape=[
            jax.ShapeDtypeStruct((NSC, N, FH), jnp.float32),
            jax.ShapeDtypeStruct((N, 1), jnp.float32),
        ],
    )(deg2, x)


def _tc_body(s2, y2, dinv, W1r, b1r, gar, Wgr, bgr, Wl1r, bl1r, Wl2r, bl2r,
             out, acc):
    i = pl.program_id(0)

    @pl.when(i == 0)
    def _():
        acc[...] = jnp.zeros_like(acc)

    dv = dinv[...]
    t = jnp.concatenate([s2[0] + y2[0], s2[1] + y2[1]], axis=1)
    agg = dv * t
    h = jnp.dot(agg, W1r[...], preferred_element_type=jnp.float32) + b1r[...]
    h = jnp.maximum(h, 0.0)
    acc[...] += jnp.sum(h, axis=0, keepdims=True)

    @pl.when(i == pl.num_programs(0) - 1)
    def _():
        hm = acc[...] / N
        g = jnp.dot(gar[...], Wgr[...], preferred_element_type=jnp.float32)
        g = jnp.maximum(g + bgr[...], 0.0)
        z = jnp.concatenate([hm, g], axis=1)
        z1 = jnp.dot(z, Wl1r[...], preferred_element_type=jnp.float32)
        z1 = jnp.maximum(z1 + bl1r[...], 0.0)
        z2 = jnp.dot(z1, Wl2r[...], preferred_element_type=jnp.float32)
        z2 = z2 + bl2r[...]
        m = jnp.max(z2, axis=1, keepdims=True)
        lse = m + jnp.log(jnp.sum(jnp.exp(z2 - m), axis=1, keepdims=True))
        out[...] = z2 - lse


def _tc_head(s2, y2, dinv2, W1, b1, ga, Wg, bg, Wl1, bl1, Wl2, bl2):
    full = lambda shape: pl.BlockSpec(shape, lambda i: tuple(0 for _ in shape))
    return pl.pallas_call(
        _tc_body,
        grid=(N // BN,),
        in_specs=[
            pl.BlockSpec((NSC, BN, FH), lambda i: (0, i, 0)),
            pl.BlockSpec((NSC, BN, FH), lambda i: (0, i, 0)),
            pl.BlockSpec((BN, 1), lambda i: (i, 0)),
            full((F_IN, H)),
            full((1, H)),
            full((1, 64)),
            full((64, H)),
            full((1, H)),
            full((2 * H, H)),
            full((1, H)),
            full((H, 2)),
            full((1, 2)),
        ],
        out_specs=pl.BlockSpec((1, 2), lambda i: (0, 0)),
        out_shape=jax.ShapeDtypeStruct((1, 2), jnp.float32),
        scratch_shapes=[pltpu.VMEM((1, H), jnp.float32)],
    )(s2, y2, dinv2, W1, b1, ga, Wg, bg, Wl1, bl1, Wl2, bl2)


def kernel(x, edge_index, edge_attr, graph_attr, W1, b1, Wg, bg, Wl1, bl1,
           Wl2, bl2):
    if graph_attr.ndim == 1:
        graph_attr = graph_attr[None, :]
    src = edge_index[0]
    dst = edge_index[1]
    deg = _sc_deg(dst, edge_attr)
    y2, dinv2 = _tc_pre(deg.reshape(N, 1), x)
    pad = E2 - E
    srcp = jnp.pad(src, (0, pad))
    dstp = jnp.pad(dst, (0, pad))
    wp = jnp.pad(edge_attr, (0, pad))
    s2 = _sc_agg(y2, srcp, dstp, wp)
    return _tc_head(s2, y2, dinv2, W1, b1.reshape(1, H),
                    graph_attr, Wg, bg.reshape(1, H), Wl1, bl1.reshape(1, H),
                    Wl2, bl2.reshape(1, 2))


# speedup vs baseline: 1.2919x; 1.0016x over previous
import jax
import jax.numpy as jnp
from jax import lax
from jax.experimental import pallas as pl
from jax.experimental.pallas import tpu as pltpu
from jax.experimental.pallas import tpu_sc as plsc

N = 10000
E = 160000
F_IN = 256
FH = 128
H = 512

NSC = 2
NT = 16
CKD = 1000
EPT = E // NT
CKA = 160
NCH = 63
EPT2 = NCH * CKA
E2 = NT * EPT2
SL = 640


def _deg_body(dst_hbm, w_hbm, deg_hbm, deg_sp, degsl, dstd, wd):
    c = lax.axis_index("c")
    s = lax.axis_index("s")
    soff = jnp.minimum(s * SL, N - SL)

    @pl.when(c == 0)
    def _():
        def _zdeg(i, _):
            degsl[pl.ds(i * 16, 16)] = jnp.zeros((16,), jnp.float32)
            return ()
        lax.fori_loop(0, SL // 16, _zdeg, ())
        pltpu.sync_copy(degsl, deg_sp.at[pl.ds(soff, SL)])
    plsc.subcore_barrier()

    @pl.when(c == 0)
    def _():
        dbase = s * EPT
        def _dchunk(g, _):
            off = dbase + g * CKD
            pltpu.sync_copy(dst_hbm.at[pl.ds(off, CKD)], dstd)
            pltpu.sync_copy(w_hbm.at[pl.ds(off, CKD)], wd)
            pltpu.sync_copy(wd, deg_sp.at[dstd], add=True)
            return ()
        lax.fori_loop(0, EPT // CKD, _dchunk, ())
    plsc.subcore_barrier()

    @pl.when(c == 0)
    def _():
        pltpu.sync_copy(deg_sp.at[pl.ds(soff, SL)], degsl)
        pltpu.sync_copy(degsl, deg_hbm.at[pl.ds(soff, SL)])


def _sc_deg(dst, w):
    mesh = plsc.VectorSubcoreMesh(core_axis_name="c", subcore_axis_name="s")
    return pl.kernel(
        _deg_body,
        out_type=[jax.ShapeDtypeStruct((N,), jnp.float32)],
        mesh=mesh,
        compiler_params=pltpu.CompilerParams(needs_layout_passes=False),
        scratch_types=[
            pltpu.VMEM_SHARED((N,), jnp.float32),
            pltpu.VMEM((SL,), jnp.float32),
            pltpu.VMEM((CKD,), jnp.int32),
            pltpu.VMEM((CKD,), jnp.float32),
        ],
    )(dst, w)[0]


def _agg_body(y2_hbm, src_hbm, dst_hbm, w_hbm, s2_hbm, S_sp,
              srca0, dsta0, wa0, rows0, gsem0, ssem0,
              srca1, dsta1, wa1, rows1, gsem1, ssem1):
    c = lax.axis_index("c")
    s = lax.axis_index("s")
    dbase = s * EPT2
    soff = jnp.minimum(s * SL, N - SL)

    def _zrow(r, _):
        for f in range(8):
            rows0[r, pl.ds(f * 16, 16)] = jnp.zeros((16,), jnp.float32)
        return ()
    lax.fori_loop(0, CKA, _zrow, ())
    for o in (0, 160, 320, 480):
        pltpu.sync_copy(rows0, S_sp.at[pl.ds(soff + o, CKA)])
    plsc.subcore_barrier()

    sets = ((srca0, dsta0, wa0, rows0, gsem0, ssem0),
            (srca1, dsta1, wa1, rows1, gsem1, ssem1))

    def _load(g, st):
        srca, dsta, wa = st[0], st[1], st[2]
        off = dbase + g * CKA
        pltpu.sync_copy(src_hbm.at[pl.ds(off, CKA)], srca)
        pltpu.sync_copy(dst_hbm.at[pl.ds(off, CKA)], dsta)
        pltpu.sync_copy(w_hbm.at[pl.ds(off, CKA)], wa)

    def _fire_gather(st):
        pltpu.async_copy(y2_hbm.at[c].at[st[0]], st[3], st[4])

    def _wait_gather(st):
        pltpu.make_async_copy(y2_hbm.at[c].at[st[0]], st[3], st[4]).wait()

    def _wait_scatter(st):
        pltpu.make_async_copy(st[3], S_sp.at[st[1]], st[5]).wait()

    def _half(g, A, B):
        _wait_gather(A)
        @pl.when(g > 0)
        def _():
            _wait_scatter(B)
        @pl.when(g + 1 < NCH)
        def _():
            _load(g + 1, B)
            _fire_gather(B)
        wa, rows = A[2], A[3]
        @plsc.parallel_loop(0, CKA, unroll=4)
        def _scale(e):
            cs = plsc.load_gather(wa, [jnp.full((16,), e, jnp.int32)])
            for f in range(8):
                sl = (e, pl.ds(f * 16, 16))
                rows[sl] = rows[sl] * cs
        pltpu.async_copy(rows, S_sp.at[A[1]], A[5], add=True)

    _load(0, sets[0])
    _fire_gather(sets[0])
    def _pair(g2, _):
        g = g2 * 2
        _half(g, sets[0], sets[1])
        _half(g + 1, sets[1], sets[0])
        return ()
    lax.fori_loop(0, (NCH - 1) // 2, _pair, ())
    _half(NCH - 1, sets[0], sets[1])
    _wait_scatter(sets[0])
    plsc.subcore_barrier()

    pltpu.sync_copy(S_sp.at[pl.ds(soff, SL)],
                    s2_hbm.at[c].at[pl.ds(soff, SL)])


def _sc_agg(y2, srcp, dstp, wp):
    mesh = plsc.VectorSubcoreMesh(core_axis_name="c", subcore_axis_name="s")
    buf = lambda: [
        pltpu.VMEM((CKA,), jnp.int32),
        pltpu.VMEM((CKA,), jnp.int32),
        pltpu.VMEM((CKA,), jnp.float32),
        pltpu.VMEM((CKA, FH), jnp.float32),
        pltpu.SemaphoreType.DMA,
        pltpu.SemaphoreType.DMA,
    ]
    return pl.kernel(
        _agg_body,
        out_type=[jax.ShapeDtypeStruct((NSC, N, FH), jnp.float32)],
        mesh=mesh,
        compiler_params=pltpu.CompilerParams(needs_layout_passes=False),
        scratch_types=[pltpu.VMEM_SHARED((N, FH), jnp.float32)]
        + buf() + buf(),
    )(y2, srcp, dstp, wp)[0]


BN = 2000


def _pre_body(deg, x, y2, dinv2):
    dv = lax.rsqrt(1.0 + deg[...])
    y = x[...] * dv
    y2[0] = y[:, :FH]
    y2[1] = y[:, FH:]
    dinv2[...] = dv


def _tc_pre(deg2, x):
    return pl.pallas_call(
        _pre_body,
        grid=(N // BN,),
        in_specs=[
            pl.BlockSpec((BN, 1), lambda i: (i, 0)),
            pl.BlockSpec((BN, F_IN), lambda i: (i, 0)),
        ],
        out_specs=[
            pl.BlockSpec((NSC, BN, FH), lambda i: (0, i, 0)),
            pl.BlockSpec((BN, 1), lambda i: (i, 0)),
        ],
        out_shape=[
            jax.ShapeDtypeStruct((NSC, N, FH), jnp.float32),
            jax.ShapeDtypeStruct((N, 1), jnp.float32),
        ],
    )(deg2, x)


def _tc_body(s2, y2, dinv, W1r, b1r, gar, Wgr, bgr, Wl1r, bl1r, Wl2r, bl2r,
             out, acc):
    i = pl.program_id(0)

    @pl.when(i == 0)
    def _():
        acc[...] = jnp.zeros_like(acc)

    dv = dinv[...]
    t = jnp.concatenate([s2[0] + y2[0], s2[1] + y2[1]], axis=1)
    agg = (dv * t).astype(jnp.bfloat16)
    h = jnp.dot(agg, W1r[...].astype(jnp.bfloat16),
                preferred_element_type=jnp.float32) + b1r[...]
    h = jnp.maximum(h, 0.0)
    acc[...] += jnp.sum(h, axis=0, keepdims=True)

    @pl.when(i == pl.num_programs(0) - 1)
    def _():
        hm = acc[...] / N
        g = jnp.dot(gar[...], Wgr[...], preferred_element_type=jnp.float32)
        g = jnp.maximum(g + bgr[...], 0.0)
        z = jnp.concatenate([hm, g], axis=1)
        z1 = jnp.dot(z, Wl1r[...], preferred_element_type=jnp.float32)
        z1 = jnp.maximum(z1 + bl1r[...], 0.0)
        z2 = jnp.dot(z1, Wl2r[...], preferred_element_type=jnp.float32)
        z2 = z2 + bl2r[...]
        m = jnp.max(z2, axis=1, keepdims=True)
        lse = m + jnp.log(jnp.sum(jnp.exp(z2 - m), axis=1, keepdims=True))
        out[...] = z2 - lse


def _tc_head(s2, y2, dinv2, W1, b1, ga, Wg, bg, Wl1, bl1, Wl2, bl2):
    full = lambda shape: pl.BlockSpec(shape, lambda i: tuple(0 for _ in shape))
    return pl.pallas_call(
        _tc_body,
        grid=(N // BN,),
        in_specs=[
            pl.BlockSpec((NSC, BN, FH), lambda i: (0, i, 0)),
            pl.BlockSpec((NSC, BN, FH), lambda i: (0, i, 0)),
            pl.BlockSpec((BN, 1), lambda i: (i, 0)),
            full((F_IN, H)),
            full((1, H)),
            full((1, 64)),
            full((64, H)),
            full((1, H)),
            full((2 * H, H)),
            full((1, H)),
            full((H, 2)),
            full((1, 2)),
        ],
        out_specs=pl.BlockSpec((1, 2), lambda i: (0, 0)),
        out_shape=jax.ShapeDtypeStruct((1, 2), jnp.float32),
        scratch_shapes=[pltpu.VMEM((1, H), jnp.float32)],
    )(s2, y2, dinv2, W1, b1, ga, Wg, bg, Wl1, bl1, Wl2, bl2)


def kernel(x, edge_index, edge_attr, graph_attr, W1, b1, Wg, bg, Wl1, bl1,
           Wl2, bl2):
    if graph_attr.ndim == 1:
        graph_attr = graph_attr[None, :]
    src = edge_index[0]
    dst = edge_index[1]
    deg = _sc_deg(dst, edge_attr)
    y2, dinv2 = _tc_pre(deg.reshape(N, 1), x)
    pad = E2 - E
    srcp = jnp.pad(src, (0, pad))
    dstp = jnp.pad(dst, (0, pad))
    wp = jnp.pad(edge_attr, (0, pad))
    s2 = _sc_agg(y2, srcp, dstp, wp)
    return _tc_head(s2, y2, dinv2, W1, b1.reshape(1, H),
                    graph_attr, Wg, bg.reshape(1, H), Wl1, bl1.reshape(1, H),
                    Wl2, bl2.reshape(1, 2))
